# baseline (speedup 1.0000x reference)
import functools

import jax
import jax.numpy as jnp
from jax import lax
from jax.experimental import pallas as pl
from jax.experimental.pallas import tpu as pltpu
from jax.experimental.pallas import tpu_sc as plsc

_VOCAB = 100000
_MAXLEN = 200
_EMBED_DIM = 64
_BATCH = 4096

_NC = 2
_NS = 16
_NW = _NC * _NS
_TOTAL = _BATCH * _MAXLEN
_ROWS_PER_W = _TOTAL // _NW
_CH = 128
_CPW = _ROWS_PER_W // _CH


@functools.partial(
    pl.kernel,
    mesh=plsc.VectorSubcoreMesh(core_axis_name="c", subcore_axis_name="s"),
    out_type=jax.ShapeDtypeStruct((_TOTAL, _EMBED_DIM), jnp.float32),
    scratch_types=[
        pltpu.VMEM((_CPW, _CH), jnp.int32),
        pltpu.VMEM((_CH, 2 * _EMBED_DIM), jnp.float32),
        pltpu.VMEM((_CH, _EMBED_DIM), jnp.float32),
        pltpu.SemaphoreType.DMA,
    ],
    compiler_params=pltpu.CompilerParams(use_tc_tiling_on_sc=True),
)
def _gather_kernel(idx_hbm, table_hbm, out_hbm, idx_v, rows_v, rows64_v, gsem):
    wid = lax.axis_index("s") * _NC + lax.axis_index("c")
    base = wid * _ROWS_PER_W
    pltpu.sync_copy(idx_hbm.at[wid], idx_v)

    def step(j, carry):
        pltpu.async_copy(table_hbm.at[idx_v.at[j]], rows_v, gsem).wait()
        def rep(r, c2):
            for k in range(_EMBED_DIM // 16):
                rows64_v[r, pl.ds(k * 16, 16)] = rows_v[r, pl.ds(k * 16, 16)]
            return c2
        lax.fori_loop(0, _CH, rep, 0)
        pltpu.sync_copy(rows64_v, out_hbm.at[pl.ds(base + j * _CH, _CH)])
        return carry

    lax.fori_loop(0, _CPW, step, 0)


def kernel(x, token_table, pos_table):
    del pos_table
    idx = x.reshape(_NW, _CPW, _CH).astype(jnp.int32)
    table_pad = jnp.pad(token_table, ((0, 0), (0, _EMBED_DIM)))
    out = _gather_kernel(idx, table_pad)
    return out.reshape(_BATCH, _MAXLEN, _EMBED_DIM)
